# baseline (speedup 1.0000x reference)
import functools

import jax
import jax.numpy as jnp
from jax import lax
from jax.experimental import pallas as pl
from jax.experimental.pallas import tpu as pltpu
from jax.experimental.pallas import tpu_sc as plsc

_NC = 2
_NS = 16
_L = 16
_NW = _NC * _NS

_RES = 256
_CHUNK = 8192
_UNROLL = 4


def _build_sc_call(n_points):
    pts_per_tile = n_points // _NW
    chunks = pts_per_tile // _CHUNK
    tbl_words = _RES * 3

    mesh = plsc.VectorSubcoreMesh(core_axis_name="c", subcore_axis_name="s")

    buf = pltpu.VMEM((_CHUNK,), jnp.float32)

    @functools.partial(
        pl.kernel,
        mesh=mesh,
        out_type=jax.ShapeDtypeStruct((6, n_points), jnp.float32),
        compiler_params=pltpu.CompilerParams(
            needs_layout_passes=False, use_tc_tiling_on_sc=True,
            skip_device_barrier=True),
        scratch_types=[
            [buf] * 4,
            [buf] * 4,
            pltpu.VMEM((tbl_words,), jnp.float32),
            pltpu.VMEM((tbl_words,), jnp.float32),
            pltpu.VMEM((tbl_words,), jnp.float32),
            pltpu.VMEM((_L,), jnp.float32),
            pltpu.SemaphoreType.DMA,
            pltpu.SemaphoreType.DMA,
            pltpu.SemaphoreType.DMA,
            pltpu.SemaphoreType.DMA,
            pltpu.SemaphoreType.DMA,
        ],
    )
    def sc_kernel(pts_hbm, g_hbm, l_hbm, a_hbm, out_hbm,
                  set_a, set_b, tbl, gbuf, lbuf, abuf,
                  isem_a, isem_b, osem_a, osem_b, csem):
        wid = lax.axis_index("s") * _NC + lax.axis_index("c")
        tile_base = wid * pts_per_tile

        sets = (set_a, set_b)
        isems = (isem_a, isem_b)
        osems = (osem_a, osem_b)

        def start_in(c):
            s = c % 2
            sl = pl.ds(tile_base + c * _CHUNK, _CHUNK)
            return [pltpu.async_copy(pts_hbm.at[3, sl], sets[s][0], isems[s])]

        def start_out_colors(c):
            s = c % 2
            sl = pl.ds(tile_base + c * _CHUNK, _CHUNK)
            return [pltpu.async_copy(sets[s][f + 1], out_hbm.at[f + 3, sl],
                                     osems[s])
                    for f in range(3)]

        in_dma = [None, None]
        out_dma = [None, None]
        in_dma[0] = start_in(0)

        tsl = pl.ds(tile_base, pts_per_tile)
        coord_dma = [pltpu.async_copy(pts_hbm.at[f, tsl], out_hbm.at[f, tsl],
                                      csem)
                     for f in range(3)]

        pltpu.sync_copy(g_hbm, gbuf)
        pltpu.sync_copy(l_hbm, lbuf)
        pltpu.sync_copy(a_hbm, abuf)
        av = abuf[...]
        bv = 1.0 - av

        def blend_body(i, _):
            sl = pl.ds(i * _L, _L)
            gi = jnp.clip(gbuf[sl], 0.0, 1.0)
            li = jnp.clip(lbuf[sl], 0.0, 1.0)
            tbl[sl] = av * gi + bv * li
            return _

        lax.fori_loop(0, tbl_words // _L, blend_body, None)

        def compute_chunk(s):
            db, rb, gb, bb = sets[s][0], sets[s][1], sets[s][2], sets[s][3]

            @plsc.parallel_loop(0, _CHUNK // _L, unroll=_UNROLL)
            def point_body(it):
                sl = pl.ds(it * _L, _L)
                ci = jnp.clip(db[sl].astype(jnp.int32), 0, _RES - 1)
                t0 = ci * 3
                rb[sl] = plsc.load_gather(tbl, [t0])
                gb[sl] = plsc.load_gather(tbl, [t0 + 1])
                bb[sl] = plsc.load_gather(tbl, [t0 + 2])

        for c in range(chunks):
            cur = c % 2
            if c + 1 < chunks:
                if c >= 1:
                    for d in out_dma[1 - cur]:
                        d.wait()
                in_dma[1 - cur] = start_in(c + 1)
            for d in in_dma[cur]:
                d.wait()
            compute_chunk(cur)
            out_dma[cur] = start_out_colors(c)

        if chunks >= 2:
            for d in out_dma[chunks % 2]:
                d.wait()
        for d in out_dma[(chunks - 1) % 2]:
            d.wait()
        for d in coord_dma:
            d.wait()

    return sc_kernel


def kernel(points, global_colors, local_colors, alpha):
    n = points.shape[0]
    pts_t = points.T
    g_flat = global_colors.reshape(-1)
    l_flat = local_colors.reshape(-1)
    alpha16 = jnp.broadcast_to(alpha.reshape(1), (_L,))
    out_t = _build_sc_call(n)(pts_t, g_flat, l_flat, alpha16)
    return out_t.T


# speedup vs baseline: 10.2822x; 10.2822x over previous
import functools

import jax
import jax.numpy as jnp
from jax import lax
from jax.experimental import pallas as pl
from jax.experimental.pallas import tpu as pltpu
from jax.experimental.pallas import tpu_sc as plsc

_NC = 2
_NS = 16
_L = 16
_NW = _NC * _NS

_RES = 256
_CHUNK = 8192
_UNROLL = 4


def _build_sc_call(n_points):
    pts_per_tile = n_points // _NW
    chunks = pts_per_tile // _CHUNK
    tbl_words = _RES * 3

    mesh = plsc.VectorSubcoreMesh(core_axis_name="c", subcore_axis_name="s")

    buf = pltpu.VMEM((_CHUNK,), jnp.float32)

    @functools.partial(
        pl.kernel,
        mesh=mesh,
        out_type=jax.ShapeDtypeStruct((6, n_points), jnp.float32),
        compiler_params=pltpu.CompilerParams(
            needs_layout_passes=False, use_tc_tiling_on_sc=True),
        scratch_types=[
            [buf] * 7,
            [buf] * 7,
            pltpu.VMEM((tbl_words,), jnp.float32),
            pltpu.VMEM((tbl_words,), jnp.float32),
            pltpu.VMEM((tbl_words,), jnp.float32),
            pltpu.VMEM((_L,), jnp.float32),
            pltpu.SemaphoreType.DMA,
            pltpu.SemaphoreType.DMA,
            pltpu.SemaphoreType.DMA,
            pltpu.SemaphoreType.DMA,
        ],
    )
    def sc_kernel(pts_hbm, g_hbm, l_hbm, a_hbm, out_hbm,
                  set_a, set_b, tbl, gbuf, lbuf, abuf,
                  isem_a, isem_b, osem_a, osem_b):
        wid = lax.axis_index("s") * _NC + lax.axis_index("c")
        tile_base = wid * pts_per_tile

        sets = (set_a, set_b)
        isems = (isem_a, isem_b)
        osems = (osem_a, osem_b)

        def start_in(c):
            s = c % 2
            sl = pl.ds(tile_base + c * _CHUNK, _CHUNK)
            return [pltpu.async_copy(pts_hbm.at[f, sl], sets[s][f], isems[s])
                    for f in range(4)]

        def start_out_coords(c):
            s = c % 2
            sl = pl.ds(tile_base + c * _CHUNK, _CHUNK)
            return [pltpu.async_copy(sets[s][f], out_hbm.at[f, sl], osems[s])
                    for f in range(3)]

        def start_out_colors(c):
            s = c % 2
            sl = pl.ds(tile_base + c * _CHUNK, _CHUNK)
            return [pltpu.async_copy(sets[s][f + 4], out_hbm.at[f + 3, sl],
                                     osems[s])
                    for f in range(3)]

        in_dma = [None, None]
        out_dma = [None, None]
        in_dma[0] = start_in(0)

        pltpu.sync_copy(g_hbm, gbuf)
        pltpu.sync_copy(l_hbm, lbuf)
        pltpu.sync_copy(a_hbm, abuf)
        av = abuf[...]
        bv = 1.0 - av

        def blend_body(i, _):
            sl = pl.ds(i * _L, _L)
            gi = jnp.clip(gbuf[sl], 0.0, 1.0)
            li = jnp.clip(lbuf[sl], 0.0, 1.0)
            tbl[sl] = av * gi + bv * li
            return _

        lax.fori_loop(0, tbl_words // _L, blend_body, None)

        def compute_chunk(s):
            db, rb, gb, bb = sets[s][3], sets[s][4], sets[s][5], sets[s][6]

            @plsc.parallel_loop(0, _CHUNK // _L, unroll=_UNROLL)
            def point_body(it):
                sl = pl.ds(it * _L, _L)
                ci = jnp.clip(db[sl].astype(jnp.int32), 0, _RES - 1)
                t0 = ci * 3
                rb[sl] = plsc.load_gather(tbl, [t0])
                gb[sl] = plsc.load_gather(tbl, [t0 + 1])
                bb[sl] = plsc.load_gather(tbl, [t0 + 2])

        for c in range(chunks):
            cur = c % 2
            if c + 1 < chunks:
                if c >= 1:
                    for d in out_dma[1 - cur]:
                        d.wait()
                in_dma[1 - cur] = start_in(c + 1)
            for d in in_dma[cur]:
                d.wait()
            out_dma[cur] = start_out_coords(c)
            compute_chunk(cur)
            out_dma[cur] += start_out_colors(c)

        if chunks >= 2:
            for d in out_dma[chunks % 2]:
                d.wait()
        for d in out_dma[(chunks - 1) % 2]:
            d.wait()

    return sc_kernel


def kernel(points, global_colors, local_colors, alpha):
    n = points.shape[0]
    pts_t = points.T
    g_flat = global_colors.reshape(-1)
    l_flat = local_colors.reshape(-1)
    alpha16 = jnp.broadcast_to(alpha.reshape(1), (_L,))
    out_t = _build_sc_call(n)(pts_t, g_flat, l_flat, alpha16)
    return out_t.T
